# baseline (speedup 1.0000x reference)
import jax
import jax.numpy as jnp
from jax import lax
from jax.experimental import pallas as pl
from jax.experimental.pallas import tpu as pltpu
from jax.experimental.pallas import tpu_sc as plsc

B, S, D, H, L, F = 16, 512, 256, 8, 6, 1024
DH = D // H
SP = 520
NTOK = 513
N = B * S
NW = 32
PER_W = N // NW
CH = 64


def _sc_gather_body(tok_hbm, abin_hbm, ttab_hbm, atab_hbm, out_t, out_a,
                    idx_t, idx_a, rows_t, rows_a, sem_t, sem_a):
    wid = lax.axis_index("s") * 2 + lax.axis_index("c")
    base = wid * PER_W
    for c in range(PER_W // CH):
        off = base + c * CH
        pltpu.sync_copy(tok_hbm.at[pl.ds(off, CH)], idx_t)
        pltpu.sync_copy(abin_hbm.at[pl.ds(off, CH)], idx_a)
        ct = pltpu.async_copy(ttab_hbm.at[idx_t], rows_t, sem_t)
        ca = pltpu.async_copy(atab_hbm.at[idx_a], rows_a, sem_a)
        ct.wait()
        ca.wait()
        pltpu.sync_copy(rows_t, out_t.at[pl.ds(off, CH)])
        pltpu.sync_copy(rows_a, out_a.at[pl.ds(off, CH)])


def _sc_gather(tok, abin, ttab, atab):
    mesh = plsc.VectorSubcoreMesh(core_axis_name="c", subcore_axis_name="s")
    k = pl.kernel(
        _sc_gather_body,
        mesh=mesh,
        out_type=[jax.ShapeDtypeStruct((N, D), jnp.float32),
                  jax.ShapeDtypeStruct((N, D), jnp.float32)],
        scratch_types=[pltpu.VMEM((CH,), jnp.int32),
                       pltpu.VMEM((CH,), jnp.int32),
                       pltpu.VMEM((CH, D), jnp.float32),
                       pltpu.VMEM((CH, D), jnp.float32),
                       pltpu.SemaphoreType.DMA,
                       pltpu.SemaphoreType.DMA],
    )
    return k(tok, abin, ttab, atab)


def _ln(x, g, b):
    m = jnp.mean(x, axis=-1, keepdims=True)
    v = jnp.mean((x - m) ** 2, axis=-1, keepdims=True)
    return (x - m) * lax.rsqrt(v + 1e-5) * g + b


def _bf(x):
    return x.astype(jnp.bfloat16)


def _mm_nt(a, b):
    return lax.dot_general(_bf(a), _bf(b), (((1,), (1,)), ((), ())),
                           preferred_element_type=jnp.float32)


def _mm_nn(a, b):
    return lax.dot_general(_bf(a), _bf(b), (((1,), (0,)), ((), ())),
                           preferred_element_type=jnp.float32)


def _tc_body(emb_t, emb_a, sv, Wqkv, bqkv, Wo, bo, ln1g, ln1b, W1, b1,
             W2, b2, ln2g, ln2b, lnfg, lnfb, out, x_ref):
    x_ref[0:S, :] = emb_t[0] + emb_a[0]
    x_ref[S:SP, :] = jnp.zeros((SP - S, D), jnp.float32)
    x_ref[S:S + 1, :] = sv[...]
    x0 = x_ref[...]
    ones8 = jnp.ones((S, 8), jnp.bfloat16)
    hseg = ((lax.broadcasted_iota(jnp.int32, (D, H), 0) // DH) ==
            lax.broadcasted_iota(jnp.int32, (D, H), 1)).astype(jnp.bfloat16)

    def layer(l, x):
        h1 = _ln(x, ln1g[l], ln1b[l])
        qkv = _mm_nt(h1, Wqkv[l]) + bqkv[l]
        qkv_b = _bf(qkv)
        ksrow = qkv[S:S + 1, D:2 * D]
        lgs = lax.dot_general(_bf(qkv[:, 0:D] * ksrow), hseg,
                              (((1,), (0,)), ((), ())),
                              preferred_element_type=jnp.float32)
        ps_all = jnp.exp2(lgs)
        heads = []
        for hh in range(H):
            q = qkv_b[:, DH * hh:DH * hh + DH]
            k512 = qkv_b[0:S, D + DH * hh:D + DH * hh + DH]
            v512 = qkv_b[0:S, 2 * D + DH * hh:2 * D + DH * hh + DH]
            p = _bf(jnp.exp2(lax.dot_general(
                q, k512, (((1,), (1,)), ((), ())),
                preferred_element_type=jnp.float32)))
            vv = jnp.concatenate([v512, ones8], axis=1)
            ov = lax.dot_general(p, vv, (((1,), (0,)), ((), ())),
                                 preferred_element_type=jnp.float32)
            ps = ps_all[:, hh:hh + 1]
            vs = qkv[S:S + 1, 2 * D + DH * hh:2 * D + DH * hh + DH]
            num = ov[:, :DH] + ps * vs
            den = ov[:, DH:DH + 1] + ps
            heads.append(num / den)
        o = jnp.concatenate(heads, axis=1)
        x = x + _mm_nt(o, Wo[l]) + bo[l]
        h2 = _ln(x, ln2g[l], ln2b[l])
        f = _mm_nt(h2, W1[l]) + b1[l]
        f = 0.5 * f * (1.0 + lax.erf(f * (2.0 ** -0.5)))
        return x + _mm_nt(f, W2[l]) + b2[l]

    x = lax.fori_loop(0, L, layer, x0, unroll=1)
    out[0] = _ln(x, lnfg[...], lnfb[...])


def _tc_forward(emb_t, emb_a, sv, Wqkv, bqkv, Wo, bo, ln1g, ln1b, W1, b1,
                W2, b2, ln2g, ln2b, lnfg, lnfb):
    def full(shape):
        ndim = len(shape)
        return pl.BlockSpec(shape, lambda b, n=ndim: (0,) * n)

    return pl.pallas_call(
        _tc_body,
        grid=(B,),
        in_specs=[
            pl.BlockSpec((1, S, D), lambda b: (b, 0, 0)),
            pl.BlockSpec((1, S, D), lambda b: (b, 0, 0)),
            full((1, D)),
            full((L, 3 * D, D)), full((L, 1, 3 * D)),
            full((L, D, D)), full((L, 1, D)),
            full((L, 1, D)), full((L, 1, D)),
            full((L, F, D)), full((L, 1, F)),
            full((L, D, F)), full((L, 1, D)),
            full((L, 1, D)), full((L, 1, D)),
            full((1, D)), full((1, D)),
        ],
        out_specs=pl.BlockSpec((1, SP, D), lambda b: (b, 0, 0)),
        out_shape=jax.ShapeDtypeStruct((B, SP, D), jnp.float32),
        scratch_shapes=[pltpu.VMEM((SP, D), jnp.float32)],
        compiler_params=pltpu.CompilerParams(
            vmem_limit_bytes=100 * 1024 * 1024),
    )(emb_t, emb_a, sv, Wqkv, bqkv, Wo, bo, ln1g, ln1b, W1, b1,
      W2, b2, ln2g, ln2b, lnfg, lnfb)


def kernel(token_ids, abund_bins, taxon_table, abund_table, sample_embed,
           Wqkv, bqkv, Wo, bo, ln1_g, ln1_b, W1, b1, W2, b2, ln2_g, ln2_b,
           lnf_g, lnf_b):
    emb_t, emb_a = _sc_gather(token_ids.reshape(N).astype(jnp.int32),
                              abund_bins.reshape(N).astype(jnp.int32),
                              taxon_table, abund_table)
    c = (DH ** -0.5) * 1.4426950408889634
    Wqkv = Wqkv.at[:, :D, :].multiply(c)
    bqkv = bqkv.at[:, :D].multiply(c)
    out = _tc_forward(
        emb_t.reshape(B, S, D), emb_a.reshape(B, S, D), sample_embed,
        Wqkv.astype(jnp.bfloat16), bqkv.reshape(L, 1, 3 * D),
        Wo.astype(jnp.bfloat16), bo.reshape(L, 1, D),
        ln1_g.reshape(L, 1, D), ln1_b.reshape(L, 1, D),
        W1.astype(jnp.bfloat16), b1.reshape(L, 1, F),
        W2.astype(jnp.bfloat16), b2.reshape(L, 1, D),
        ln2_g.reshape(L, 1, D), ln2_b.reshape(L, 1, D),
        lnf_g.reshape(1, D), lnf_b.reshape(1, D))
    h = jnp.concatenate([out[:, S:S + 1, :], out[:, 0:S, :]], axis=1)
    return (h, h[:, 0, :])


# speedup vs baseline: 1.0112x; 1.0112x over previous
import jax
import jax.numpy as jnp
from jax import lax
from jax.experimental import pallas as pl
from jax.experimental.pallas import tpu as pltpu
from jax.experimental.pallas import tpu_sc as plsc

B, S, D, H, L, F = 16, 512, 256, 8, 6, 1024
DH = D // H
SP = 520
NTOK = 513
N = B * S
NW = 32
PER_W = N // NW
CH = 64
PB = 1


def _sc_gather_body(tok_hbm, abin_hbm, ttab_hbm, atab_hbm, out_t, out_a,
                    idx_t, idx_a, rows_t, rows_a, sem_t, sem_a):
    wid = lax.axis_index("s") * 2 + lax.axis_index("c")
    base = wid * PER_W
    for c in range(PER_W // CH):
        off = base + c * CH
        pltpu.sync_copy(tok_hbm.at[pl.ds(off, CH)], idx_t)
        pltpu.sync_copy(abin_hbm.at[pl.ds(off, CH)], idx_a)
        ct = pltpu.async_copy(ttab_hbm.at[idx_t], rows_t, sem_t)
        ca = pltpu.async_copy(atab_hbm.at[idx_a], rows_a, sem_a)
        ct.wait()
        ca.wait()
        pltpu.sync_copy(rows_t, out_t.at[pl.ds(off, CH)])
        pltpu.sync_copy(rows_a, out_a.at[pl.ds(off, CH)])


def _sc_gather(tok, abin, ttab, atab):
    mesh = plsc.VectorSubcoreMesh(core_axis_name="c", subcore_axis_name="s")
    k = pl.kernel(
        _sc_gather_body,
        mesh=mesh,
        out_type=[jax.ShapeDtypeStruct((N, D), jnp.float32),
                  jax.ShapeDtypeStruct((N, D), jnp.float32)],
        scratch_types=[pltpu.VMEM((CH,), jnp.int32),
                       pltpu.VMEM((CH,), jnp.int32),
                       pltpu.VMEM((CH, D), jnp.float32),
                       pltpu.VMEM((CH, D), jnp.float32),
                       pltpu.SemaphoreType.DMA,
                       pltpu.SemaphoreType.DMA],
    )
    return k(tok, abin, ttab, atab)


def _ln(x, g, b):
    m = jnp.mean(x, axis=-1, keepdims=True)
    v = jnp.mean((x - m) ** 2, axis=-1, keepdims=True)
    return (x - m) * lax.rsqrt(v + 1e-5) * g + b


def _bf(x):
    return x.astype(jnp.bfloat16)


def _mm_nt(a, b):
    return lax.dot_general(_bf(a), _bf(b), (((1,), (1,)), ((), ())),
                           preferred_element_type=jnp.float32)


def _mm_nn(a, b):
    return lax.dot_general(_bf(a), _bf(b), (((1,), (0,)), ((), ())),
                           preferred_element_type=jnp.float32)


def _tc_body(emb_t, emb_a, sv, Wqkv, bqkv, Wo, bo, ln1g, ln1b, W1, b1,
             W2, b2, ln2g, ln2b, lnfg, lnfb, out, x_ref):
    for s in range(PB):
        x_ref[s * SP:s * SP + S, :] = emb_t[s] + emb_a[s]
        x_ref[s * SP + S:(s + 1) * SP, :] = jnp.zeros((SP - S, D), jnp.float32)
        x_ref[s * SP + S:s * SP + S + 1, :] = sv[...]
    x0 = x_ref[...]
    ones8 = jnp.ones((S, 8), jnp.bfloat16)
    hseg = ((lax.broadcasted_iota(jnp.int32, (D, H), 0) // DH) ==
            lax.broadcasted_iota(jnp.int32, (D, H), 1)).astype(jnp.bfloat16)

    def layer(l, x):
        h1 = _ln(x, ln1g[l], ln1b[l])
        qkv = _mm_nt(h1, Wqkv[l]) + bqkv[l]
        qkv_b = _bf(qkv)
        ps_all = []
        for s in range(PB):
            ksrow = qkv[s * SP + S:s * SP + S + 1, D:2 * D]
            lgs = lax.dot_general(
                _bf(qkv[s * SP:(s + 1) * SP, 0:D] * ksrow), hseg,
                (((1,), (0,)), ((), ())), preferred_element_type=jnp.float32)
            ps_all.append(jnp.exp2(lgs))
        chunks = []
        for s in range(PB):
            heads = []
            for hh in range(H):
                q = qkv_b[s * SP:(s + 1) * SP, DH * hh:DH * hh + DH]
                k512 = qkv_b[s * SP:s * SP + S, D + DH * hh:D + DH * hh + DH]
                v512 = qkv_b[s * SP:s * SP + S,
                             2 * D + DH * hh:2 * D + DH * hh + DH]
                p = _bf(jnp.exp2(lax.dot_general(
                    q, k512, (((1,), (1,)), ((), ())),
                    preferred_element_type=jnp.float32)))
                vv = jnp.concatenate([v512, ones8], axis=1)
                ov = lax.dot_general(p, vv, (((1,), (0,)), ((), ())),
                                     preferred_element_type=jnp.float32)
                ps = ps_all[s][:, hh:hh + 1]
                vs = qkv[s * SP + S:s * SP + S + 1,
                         2 * D + DH * hh:2 * D + DH * hh + DH]
                num = ov[:, :DH] + ps * vs
                den = ov[:, DH:DH + 1] + ps
                heads.append(num / den)
            chunks.append(jnp.concatenate(heads, axis=1))
        o = jnp.concatenate(chunks, axis=0)
        x = x + _mm_nt(o, Wo[l]) + bo[l]
        h2 = _ln(x, ln2g[l], ln2b[l])
        f = _mm_nt(h2, W1[l]) + b1[l]
        f = 0.5 * f * (1.0 + lax.erf(f * (2.0 ** -0.5)))
        return x + _mm_nt(f, W2[l]) + b2[l]

    x = lax.fori_loop(0, L, layer, x0, unroll=1)
    xf = _ln(x, lnfg[...], lnfb[...])
    for s in range(PB):
        out[s, 0:1, :] = xf[s * SP + S:s * SP + S + 1, :]
        out[s, 1:NTOK, :] = xf[s * SP:s * SP + S, :]


def _tc_forward(emb_t, emb_a, sv, Wqkv, bqkv, Wo, bo, ln1g, ln1b, W1, b1,
                W2, b2, ln2g, ln2b, lnfg, lnfb):
    def full(shape):
        ndim = len(shape)
        return pl.BlockSpec(shape, lambda b, n=ndim: (0,) * n)

    return pl.pallas_call(
        _tc_body,
        grid=(B // PB,),
        in_specs=[
            pl.BlockSpec((PB, S, D), lambda b: (b, 0, 0)),
            pl.BlockSpec((PB, S, D), lambda b: (b, 0, 0)),
            full((1, D)),
            full((L, 3 * D, D)), full((L, 1, 3 * D)),
            full((L, D, D)), full((L, 1, D)),
            full((L, 1, D)), full((L, 1, D)),
            full((L, F, D)), full((L, 1, F)),
            full((L, D, F)), full((L, 1, D)),
            full((L, 1, D)), full((L, 1, D)),
            full((1, D)), full((1, D)),
        ],
        out_specs=pl.BlockSpec((PB, NTOK, D), lambda b: (b, 0, 0)),
        out_shape=jax.ShapeDtypeStruct((B, NTOK, D), jnp.float32),
        scratch_shapes=[pltpu.VMEM((PB * SP, D), jnp.float32)],
        compiler_params=pltpu.CompilerParams(
            vmem_limit_bytes=100 * 1024 * 1024),
    )(emb_t, emb_a, sv, Wqkv, bqkv, Wo, bo, ln1g, ln1b, W1, b1,
      W2, b2, ln2g, ln2b, lnfg, lnfb)


def kernel(token_ids, abund_bins, taxon_table, abund_table, sample_embed,
           Wqkv, bqkv, Wo, bo, ln1_g, ln1_b, W1, b1, W2, b2, ln2_g, ln2_b,
           lnf_g, lnf_b):
    emb_t, emb_a = _sc_gather(token_ids.reshape(N).astype(jnp.int32),
                              abund_bins.reshape(N).astype(jnp.int32),
                              taxon_table, abund_table)
    c = (DH ** -0.5) * 1.4426950408889634
    Wqkv = Wqkv.at[:, :D, :].multiply(c)
    bqkv = bqkv.at[:, :D].multiply(c)
    out = _tc_forward(
        emb_t.reshape(B, S, D), emb_a.reshape(B, S, D), sample_embed,
        Wqkv.astype(jnp.bfloat16), bqkv.reshape(L, 1, 3 * D),
        Wo.astype(jnp.bfloat16), bo.reshape(L, 1, D),
        ln1_g.reshape(L, 1, D), ln1_b.reshape(L, 1, D),
        W1.astype(jnp.bfloat16), b1.reshape(L, 1, F),
        W2.astype(jnp.bfloat16), b2.reshape(L, 1, D),
        ln2_g.reshape(L, 1, D), ln2_b.reshape(L, 1, D),
        lnf_g.reshape(1, D), lnf_b.reshape(1, D))
    return (out, out[:, 0, :])


# speedup vs baseline: 1.0910x; 1.0789x over previous
import jax
import jax.numpy as jnp
from jax import lax
from jax.experimental import pallas as pl
from jax.experimental.pallas import tpu as pltpu
from jax.experimental.pallas import tpu_sc as plsc

B, S, D, H, L, F = 16, 512, 256, 8, 6, 1024
DH = D // H
SP = 520
NTOK = 513
N = B * S
NW = 32
PER_W = N // NW
CH = 64
PB = 2


def _sc_gather_body(tok_hbm, abin_hbm, ttab_hbm, atab_hbm, out_t, out_a,
                    idx_t, idx_a, rows_t, rows_a, sem_t, sem_a):
    wid = lax.axis_index("s") * 2 + lax.axis_index("c")
    base = wid * PER_W
    for c in range(PER_W // CH):
        off = base + c * CH
        pltpu.sync_copy(tok_hbm.at[pl.ds(off, CH)], idx_t)
        pltpu.sync_copy(abin_hbm.at[pl.ds(off, CH)], idx_a)
        ct = pltpu.async_copy(ttab_hbm.at[idx_t], rows_t, sem_t)
        ca = pltpu.async_copy(atab_hbm.at[idx_a], rows_a, sem_a)
        ct.wait()
        ca.wait()
        pltpu.sync_copy(rows_t, out_t.at[pl.ds(off, CH)])
        pltpu.sync_copy(rows_a, out_a.at[pl.ds(off, CH)])


def _sc_gather(tok, abin, ttab, atab):
    mesh = plsc.VectorSubcoreMesh(core_axis_name="c", subcore_axis_name="s")
    k = pl.kernel(
        _sc_gather_body,
        mesh=mesh,
        out_type=[jax.ShapeDtypeStruct((N, D), jnp.float32),
                  jax.ShapeDtypeStruct((N, D), jnp.float32)],
        scratch_types=[pltpu.VMEM((CH,), jnp.int32),
                       pltpu.VMEM((CH,), jnp.int32),
                       pltpu.VMEM((CH, D), jnp.float32),
                       pltpu.VMEM((CH, D), jnp.float32),
                       pltpu.SemaphoreType.DMA,
                       pltpu.SemaphoreType.DMA],
    )
    return k(tok, abin, ttab, atab)


def _ln(x, g, b):
    m = jnp.mean(x, axis=-1, keepdims=True)
    v = jnp.mean((x - m) ** 2, axis=-1, keepdims=True)
    return (x - m) * lax.rsqrt(v + 1e-5) * g + b


def _bf(x):
    return x.astype(jnp.bfloat16)


def _mm_nt(a, b):
    return lax.dot_general(_bf(a), _bf(b), (((1,), (1,)), ((), ())),
                           preferred_element_type=jnp.float32)


def _mm_nn(a, b):
    return lax.dot_general(_bf(a), _bf(b), (((1,), (0,)), ((), ())),
                           preferred_element_type=jnp.float32)


def _tc_body(emb_t, emb_a, sv, Wqkv, bqkv, Wo, bo, ln1g, ln1b, W1, b1,
             W2, b2, ln2g, ln2b, lnfg, lnfb, out, x_ref):
    for s in range(PB):
        x_ref[s * SP:s * SP + S, :] = emb_t[s] + emb_a[s]
        x_ref[s * SP + S:(s + 1) * SP, :] = jnp.zeros((SP - S, D), jnp.float32)
        x_ref[s * SP + S:s * SP + S + 1, :] = sv[...]
    x0 = tuple(x_ref[s * SP:(s + 1) * SP, :] for s in range(PB))
    ones8 = jnp.ones((S, 8), jnp.bfloat16)
    hseg = ((lax.broadcasted_iota(jnp.int32, (D, H), 0) // DH) ==
            lax.broadcasted_iota(jnp.int32, (D, H), 1)).astype(jnp.bfloat16)

    def one_sample(xs, Wq_, bq_, Wo_, bo_, ln1g_, ln1b_, W1_, b1_, W2_,
                   b2_, ln2g_, ln2b_):
        h1 = _ln(xs, ln1g_, ln1b_)
        qkv = _mm_nt(h1, Wq_) + bq_
        qkv_b = _bf(qkv)
        ksrow = qkv[S:S + 1, D:2 * D]
        lgs = lax.dot_general(_bf(qkv[:, 0:D] * ksrow), hseg,
                              (((1,), (0,)), ((), ())),
                              preferred_element_type=jnp.float32)
        ps_all = jnp.exp2(lgs)
        heads = []
        for hh in range(H):
            q = qkv_b[:, DH * hh:DH * hh + DH]
            k512 = qkv_b[0:S, D + DH * hh:D + DH * hh + DH]
            v512 = qkv_b[0:S, 2 * D + DH * hh:2 * D + DH * hh + DH]
            p = _bf(jnp.exp2(lax.dot_general(
                q, k512, (((1,), (1,)), ((), ())),
                preferred_element_type=jnp.float32)))
            vv = jnp.concatenate([v512, ones8], axis=1)
            ov = lax.dot_general(p, vv, (((1,), (0,)), ((), ())),
                                 preferred_element_type=jnp.float32)
            ps = ps_all[:, hh:hh + 1]
            vs = qkv[S:S + 1, 2 * D + DH * hh:2 * D + DH * hh + DH]
            num = ov[:, :DH] + ps * vs
            den = ov[:, DH:DH + 1] + ps
            heads.append(num / den)
        o = jnp.concatenate(heads, axis=1)
        xs = xs + _mm_nt(o, Wo_) + bo_
        h2 = _ln(xs, ln2g_, ln2b_)
        f = _mm_nt(h2, W1_) + b1_
        f = 0.5 * f * (1.0 + lax.erf(f * (2.0 ** -0.5)))
        return xs + _mm_nt(f, W2_) + b2_

    def layer(l, xc):
        w = (Wqkv[l], bqkv[l], Wo[l], bo[l], ln1g[l], ln1b[l],
             W1[l], b1[l], W2[l], b2[l], ln2g[l], ln2b[l])
        return tuple(one_sample(xs, *w) for xs in xc)

    xc = lax.fori_loop(0, L, layer, x0, unroll=1)
    for s in range(PB):
        xf = _ln(xc[s], lnfg[...], lnfb[...])
        out[s, 0:1, :] = xf[S:S + 1, :]
        out[s, 1:NTOK, :] = xf[0:S, :]


def _tc_forward(emb_t, emb_a, sv, Wqkv, bqkv, Wo, bo, ln1g, ln1b, W1, b1,
                W2, b2, ln2g, ln2b, lnfg, lnfb):
    def full(shape):
        ndim = len(shape)
        return pl.BlockSpec(shape, lambda b, n=ndim: (0,) * n)

    return pl.pallas_call(
        _tc_body,
        grid=(B // PB,),
        in_specs=[
            pl.BlockSpec((PB, S, D), lambda b: (b, 0, 0)),
            pl.BlockSpec((PB, S, D), lambda b: (b, 0, 0)),
            full((1, D)),
            full((L, 3 * D, D)), full((L, 1, 3 * D)),
            full((L, D, D)), full((L, 1, D)),
            full((L, 1, D)), full((L, 1, D)),
            full((L, F, D)), full((L, 1, F)),
            full((L, D, F)), full((L, 1, D)),
            full((L, 1, D)), full((L, 1, D)),
            full((1, D)), full((1, D)),
        ],
        out_specs=pl.BlockSpec((PB, NTOK, D), lambda b: (b, 0, 0)),
        out_shape=jax.ShapeDtypeStruct((B, NTOK, D), jnp.float32),
        scratch_shapes=[pltpu.VMEM((PB * SP, D), jnp.float32)],
        compiler_params=pltpu.CompilerParams(
            vmem_limit_bytes=100 * 1024 * 1024),
    )(emb_t, emb_a, sv, Wqkv, bqkv, Wo, bo, ln1g, ln1b, W1, b1,
      W2, b2, ln2g, ln2b, lnfg, lnfb)


def kernel(token_ids, abund_bins, taxon_table, abund_table, sample_embed,
           Wqkv, bqkv, Wo, bo, ln1_g, ln1_b, W1, b1, W2, b2, ln2_g, ln2_b,
           lnf_g, lnf_b):
    emb_t, emb_a = _sc_gather(token_ids.reshape(N).astype(jnp.int32),
                              abund_bins.reshape(N).astype(jnp.int32),
                              taxon_table, abund_table)
    c = (DH ** -0.5) * 1.4426950408889634
    Wqkv = Wqkv.at[:, :D, :].multiply(c)
    bqkv = bqkv.at[:, :D].multiply(c)
    out = _tc_forward(
        emb_t.reshape(B, S, D), emb_a.reshape(B, S, D), sample_embed,
        Wqkv.astype(jnp.bfloat16), bqkv.reshape(L, 1, 3 * D),
        Wo.astype(jnp.bfloat16), bo.reshape(L, 1, D),
        ln1_g.reshape(L, 1, D), ln1_b.reshape(L, 1, D),
        W1.astype(jnp.bfloat16), b1.reshape(L, 1, F),
        W2.astype(jnp.bfloat16), b2.reshape(L, 1, D),
        ln2_g.reshape(L, 1, D), ln2_b.reshape(L, 1, D),
        lnf_g.reshape(1, D), lnf_b.reshape(1, D))
    return (out, out[:, 0, :])


# speedup vs baseline: 1.1097x; 1.0171x over previous
import jax
import jax.numpy as jnp
from jax import lax
from jax.experimental import pallas as pl
from jax.experimental.pallas import tpu as pltpu
from jax.experimental.pallas import tpu_sc as plsc

B, S, D, H, L, F = 16, 512, 256, 8, 6, 1024
DH = D // H
SP = 520
NTOK = 513
N = B * S
NW = 32
PER_W = N // NW
CH = 64
PB = 4


def _sc_gather_body(tok_hbm, abin_hbm, ttab_hbm, atab_hbm, out_t, out_a,
                    idx_t, idx_a, rows_t, rows_a, sem_t, sem_a):
    wid = lax.axis_index("s") * 2 + lax.axis_index("c")
    base = wid * PER_W
    for c in range(PER_W // CH):
        off = base + c * CH
        pltpu.sync_copy(tok_hbm.at[pl.ds(off, CH)], idx_t)
        pltpu.sync_copy(abin_hbm.at[pl.ds(off, CH)], idx_a)
        ct = pltpu.async_copy(ttab_hbm.at[idx_t], rows_t, sem_t)
        ca = pltpu.async_copy(atab_hbm.at[idx_a], rows_a, sem_a)
        ct.wait()
        ca.wait()
        pltpu.sync_copy(rows_t, out_t.at[pl.ds(off, CH)])
        pltpu.sync_copy(rows_a, out_a.at[pl.ds(off, CH)])


def _sc_gather(tok, abin, ttab, atab):
    mesh = plsc.VectorSubcoreMesh(core_axis_name="c", subcore_axis_name="s")
    k = pl.kernel(
        _sc_gather_body,
        mesh=mesh,
        out_type=[jax.ShapeDtypeStruct((N, D), jnp.float32),
                  jax.ShapeDtypeStruct((N, D), jnp.float32)],
        scratch_types=[pltpu.VMEM((CH,), jnp.int32),
                       pltpu.VMEM((CH,), jnp.int32),
                       pltpu.VMEM((CH, D), jnp.float32),
                       pltpu.VMEM((CH, D), jnp.float32),
                       pltpu.SemaphoreType.DMA,
                       pltpu.SemaphoreType.DMA],
    )
    return k(tok, abin, ttab, atab)


def _ln(x, g, b):
    m = jnp.mean(x, axis=-1, keepdims=True)
    v = jnp.mean((x - m) ** 2, axis=-1, keepdims=True)
    return (x - m) * lax.rsqrt(v + 1e-5) * g + b


def _bf(x):
    return x.astype(jnp.bfloat16)


def _mm_nt(a, b):
    return lax.dot_general(_bf(a), _bf(b), (((1,), (1,)), ((), ())),
                           preferred_element_type=jnp.float32)


def _mm_nn(a, b):
    return lax.dot_general(_bf(a), _bf(b), (((1,), (0,)), ((), ())),
                           preferred_element_type=jnp.float32)


def _tc_body(emb_t, emb_a, sv, Wqkv, bqkv, Wo, bo, ln1g, ln1b, W1, b1,
             W2, b2, ln2g, ln2b, lnfg, lnfb, out, x_ref):
    for s in range(PB):
        x_ref[s * SP:s * SP + S, :] = emb_t[s] + emb_a[s]
        x_ref[s * SP + S:(s + 1) * SP, :] = jnp.zeros((SP - S, D), jnp.float32)
        x_ref[s * SP + S:s * SP + S + 1, :] = sv[...]
    x0 = tuple(x_ref[s * SP:(s + 1) * SP, :] for s in range(PB))
    ones8 = jnp.ones((S, 8), jnp.bfloat16)
    hseg = ((lax.broadcasted_iota(jnp.int32, (D, H), 0) // DH) ==
            lax.broadcasted_iota(jnp.int32, (D, H), 1)).astype(jnp.bfloat16)

    def one_sample(xs, Wq_, bq_, Wo_, bo_, ln1g_, ln1b_, W1_, b1_, W2_,
                   b2_, ln2g_, ln2b_):
        h1 = _ln(xs, ln1g_, ln1b_)
        qkv = _mm_nt(h1, Wq_) + bq_
        qkv_b = _bf(qkv)
        ksrow = qkv[S:S + 1, D:2 * D]
        lgs = lax.dot_general(_bf(qkv[:, 0:D] * ksrow), hseg,
                              (((1,), (0,)), ((), ())),
                              preferred_element_type=jnp.float32)
        ps_all = jnp.exp2(lgs)
        heads = []
        for hh in range(H):
            q = qkv_b[:, DH * hh:DH * hh + DH]
            k512 = qkv_b[0:S, D + DH * hh:D + DH * hh + DH]
            v512 = qkv_b[0:S, 2 * D + DH * hh:2 * D + DH * hh + DH]
            p = _bf(jnp.exp2(lax.dot_general(
                q, k512, (((1,), (1,)), ((), ())),
                preferred_element_type=jnp.float32)))
            vv = jnp.concatenate([v512, ones8], axis=1)
            ov = lax.dot_general(p, vv, (((1,), (0,)), ((), ())),
                                 preferred_element_type=jnp.float32)
            ps = ps_all[:, hh:hh + 1]
            vs = qkv[S:S + 1, 2 * D + DH * hh:2 * D + DH * hh + DH]
            num = ov[:, :DH] + ps * vs
            den = ov[:, DH:DH + 1] + ps
            heads.append(num / den)
        o = jnp.concatenate(heads, axis=1)
        xs = xs + _mm_nt(o, Wo_) + bo_
        h2 = _ln(xs, ln2g_, ln2b_)
        f = _mm_nt(h2, W1_) + b1_
        f = 0.5 * f * (1.0 + lax.erf(f * (2.0 ** -0.5)))
        return xs + _mm_nt(f, W2_) + b2_

    def layer(l, xc):
        w = (Wqkv[l], bqkv[l], Wo[l], bo[l], ln1g[l], ln1b[l],
             W1[l], b1[l], W2[l], b2[l], ln2g[l], ln2b[l])
        return tuple(one_sample(xs, *w) for xs in xc)

    xc = lax.fori_loop(0, L, layer, x0, unroll=1)
    for s in range(PB):
        xf = _ln(xc[s], lnfg[...], lnfb[...])
        out[s, 0:1, :] = xf[S:S + 1, :]
        out[s, 1:NTOK, :] = xf[0:S, :]


def _tc_forward(emb_t, emb_a, sv, Wqkv, bqkv, Wo, bo, ln1g, ln1b, W1, b1,
                W2, b2, ln2g, ln2b, lnfg, lnfb):
    def full(shape):
        ndim = len(shape)
        return pl.BlockSpec(shape, lambda b, n=ndim: (0,) * n)

    return pl.pallas_call(
        _tc_body,
        grid=(B // PB,),
        in_specs=[
            pl.BlockSpec((PB, S, D), lambda b: (b, 0, 0)),
            pl.BlockSpec((PB, S, D), lambda b: (b, 0, 0)),
            full((1, D)),
            full((L, 3 * D, D)), full((L, 1, 3 * D)),
            full((L, D, D)), full((L, 1, D)),
            full((L, 1, D)), full((L, 1, D)),
            full((L, F, D)), full((L, 1, F)),
            full((L, D, F)), full((L, 1, D)),
            full((L, 1, D)), full((L, 1, D)),
            full((1, D)), full((1, D)),
        ],
        out_specs=pl.BlockSpec((PB, NTOK, D), lambda b: (b, 0, 0)),
        out_shape=jax.ShapeDtypeStruct((B, NTOK, D), jnp.float32),
        scratch_shapes=[pltpu.VMEM((PB * SP, D), jnp.float32)],
        compiler_params=pltpu.CompilerParams(
            vmem_limit_bytes=100 * 1024 * 1024),
    )(emb_t, emb_a, sv, Wqkv, bqkv, Wo, bo, ln1g, ln1b, W1, b1,
      W2, b2, ln2g, ln2b, lnfg, lnfb)


def kernel(token_ids, abund_bins, taxon_table, abund_table, sample_embed,
           Wqkv, bqkv, Wo, bo, ln1_g, ln1_b, W1, b1, W2, b2, ln2_g, ln2_b,
           lnf_g, lnf_b):
    emb_t, emb_a = _sc_gather(token_ids.reshape(N).astype(jnp.int32),
                              abund_bins.reshape(N).astype(jnp.int32),
                              taxon_table, abund_table)
    c = (DH ** -0.5) * 1.4426950408889634
    Wqkv = Wqkv.at[:, :D, :].multiply(c)
    bqkv = bqkv.at[:, :D].multiply(c)
    out = _tc_forward(
        emb_t.reshape(B, S, D), emb_a.reshape(B, S, D), sample_embed,
        Wqkv.astype(jnp.bfloat16), bqkv.reshape(L, 1, 3 * D),
        Wo.astype(jnp.bfloat16), bo.reshape(L, 1, D),
        ln1_g.reshape(L, 1, D), ln1_b.reshape(L, 1, D),
        W1.astype(jnp.bfloat16), b1.reshape(L, 1, F),
        W2.astype(jnp.bfloat16), b2.reshape(L, 1, D),
        ln2_g.reshape(L, 1, D), ln2_b.reshape(L, 1, D),
        lnf_g.reshape(1, D), lnf_b.reshape(1, D))
    return (out, out[:, 0, :])


# speedup vs baseline: 1.2484x; 1.1250x over previous
import jax
import jax.numpy as jnp
from jax import lax
from jax.experimental import pallas as pl
from jax.experimental.pallas import tpu as pltpu
from jax.experimental.pallas import tpu_sc as plsc

B, S, D, H, L, F = 16, 512, 256, 8, 6, 1024
DH = D // H
SP = 520
NTOK = 513
N = B * S
NW = 32
PER_W = N // NW
CH = 64
PB = 4


def _sc_gather_body(tok_hbm, abin_hbm, ttab_hbm, atab_hbm, out_t, out_a,
                    idx_t, idx_a, rows_t, rows_a, sem_t, sem_a):
    wid = lax.axis_index("s") * 2 + lax.axis_index("c")
    base = wid * PER_W
    for c in range(PER_W // CH):
        off = base + c * CH
        pltpu.sync_copy(tok_hbm.at[pl.ds(off, CH)], idx_t)
        pltpu.sync_copy(abin_hbm.at[pl.ds(off, CH)], idx_a)
        ct = pltpu.async_copy(ttab_hbm.at[idx_t], rows_t, sem_t)
        ca = pltpu.async_copy(atab_hbm.at[idx_a], rows_a, sem_a)
        ct.wait()
        ca.wait()
        pltpu.sync_copy(rows_t, out_t.at[pl.ds(off, CH)])
        pltpu.sync_copy(rows_a, out_a.at[pl.ds(off, CH)])


def _sc_gather(tok, abin, ttab, atab):
    mesh = plsc.VectorSubcoreMesh(core_axis_name="c", subcore_axis_name="s")
    k = pl.kernel(
        _sc_gather_body,
        mesh=mesh,
        out_type=[jax.ShapeDtypeStruct((N, D), jnp.float32),
                  jax.ShapeDtypeStruct((N, D), jnp.float32)],
        scratch_types=[pltpu.VMEM((CH,), jnp.int32),
                       pltpu.VMEM((CH,), jnp.int32),
                       pltpu.VMEM((CH, D), jnp.float32),
                       pltpu.VMEM((CH, D), jnp.float32),
                       pltpu.SemaphoreType.DMA,
                       pltpu.SemaphoreType.DMA],
    )
    return k(tok, abin, ttab, atab)


def _ln(x, g, b):
    m = jnp.mean(x, axis=-1, keepdims=True)
    v = jnp.mean((x - m) ** 2, axis=-1, keepdims=True)
    return (x - m) * lax.rsqrt(v + 1e-5) * g + b


def _bf(x):
    return x.astype(jnp.bfloat16)


def _mm_nt(a, b):
    return lax.dot_general(_bf(a), _bf(b), (((1,), (1,)), ((), ())),
                           preferred_element_type=jnp.float32)


def _mm_nn(a, b):
    return lax.dot_general(_bf(a), _bf(b), (((1,), (0,)), ((), ())),
                           preferred_element_type=jnp.float32)


def _tc_body(emb_t, emb_a, sv, Wqkv, bqkv, Wo, bo, ln1g, ln1b, W1, b1,
             W2, b2, ln2g, ln2b, lnfg, lnfb, out, x_ref):
    for s in range(PB):
        x_ref[s * SP:s * SP + S, :] = emb_t[s] + emb_a[s]
        x_ref[s * SP + S:(s + 1) * SP, :] = jnp.zeros((SP - S, D), jnp.float32)
        x_ref[s * SP + S:s * SP + S + 1, :] = sv[...]
    x0 = tuple(x_ref[s * SP:(s + 1) * SP, :] for s in range(PB))
    ones8 = jnp.ones((S, 8), jnp.bfloat16)
    hseg = ((lax.broadcasted_iota(jnp.int32, (D, H), 0) // DH) ==
            lax.broadcasted_iota(jnp.int32, (D, H), 1)).astype(jnp.bfloat16)
    hexp = ((lax.broadcasted_iota(jnp.int32, (H, D), 0) ==
             lax.broadcasted_iota(jnp.int32, (H, D), 1) // DH)
            ).astype(jnp.bfloat16)

    def one_sample(xs, Wq_, bq_, Wo_, bo_, ln1g_, ln1b_, W1_, b1_, W2_,
                   b2_, ln2g_, ln2b_):
        h1 = _ln(xs, ln1g_, ln1b_)
        qkv = _mm_nt(h1, Wq_) + bq_
        qkv_b = _bf(qkv)
        ksrow = qkv[S:S + 1, D:2 * D]
        lgs = lax.dot_general(_bf(qkv[:, 0:D] * ksrow), hseg,
                              (((1,), (0,)), ((), ())),
                              preferred_element_type=jnp.float32)
        ps_all = jnp.exp2(lgs)
        ovs = []
        for hh in range(H):
            q = qkv_b[:, DH * hh:DH * hh + DH]
            k512 = qkv_b[0:S, D + DH * hh:D + DH * hh + DH]
            v512 = qkv_b[0:S, 2 * D + DH * hh:2 * D + DH * hh + DH]
            p = _bf(jnp.exp2(lax.dot_general(
                q, k512, (((1,), (1,)), ((), ())),
                preferred_element_type=jnp.float32)))
            vv = jnp.concatenate([v512, ones8], axis=1)
            ovs.append(lax.dot_general(p, vv, (((1,), (0,)), ((), ())),
                                       preferred_element_type=jnp.float32))
        ovn = jnp.concatenate([ov[:, :DH] for ov in ovs], axis=1)
        den8 = jnp.concatenate([ov[:, DH:DH + 1] for ov in ovs],
                               axis=1) + ps_all
        re = lax.dot_general(_bf(1.0 / den8), hexp, (((1,), (0,)), ((), ())),
                             preferred_element_type=jnp.float32)
        pse = lax.dot_general(_bf(ps_all), hexp, (((1,), (0,)), ((), ())),
                              preferred_element_type=jnp.float32)
        vsrow = qkv[S:S + 1, 2 * D:3 * D]
        o = (ovn + pse * vsrow) * re
        xs = xs + _mm_nt(o, Wo_) + bo_
        h2 = _ln(xs, ln2g_, ln2b_)
        f = _mm_nt(h2, W1_) + b1_
        f = 0.5 * f * (1.0 + lax.erf(f * (2.0 ** -0.5)))
        return xs + _mm_nt(f, W2_) + b2_

    def layer(l, xc):
        w = (Wqkv[l], bqkv[l], Wo[l], bo[l], ln1g[l], ln1b[l],
             W1[l], b1[l], W2[l], b2[l], ln2g[l], ln2b[l])
        return tuple(one_sample(xs, *w) for xs in xc)

    xc = lax.fori_loop(0, L, layer, x0, unroll=1)
    for s in range(PB):
        xf = _ln(xc[s], lnfg[...], lnfb[...])
        out[s, 0:1, :] = xf[S:S + 1, :]
        out[s, 1:NTOK, :] = xf[0:S, :]


def _tc_forward(emb_t, emb_a, sv, Wqkv, bqkv, Wo, bo, ln1g, ln1b, W1, b1,
                W2, b2, ln2g, ln2b, lnfg, lnfb):
    def full(shape):
        ndim = len(shape)
        return pl.BlockSpec(shape, lambda b, n=ndim: (0,) * n)

    return pl.pallas_call(
        _tc_body,
        grid=(B // PB,),
        in_specs=[
            pl.BlockSpec((PB, S, D), lambda b: (b, 0, 0)),
            pl.BlockSpec((PB, S, D), lambda b: (b, 0, 0)),
            full((1, D)),
            full((L, 3 * D, D)), full((L, 1, 3 * D)),
            full((L, D, D)), full((L, 1, D)),
            full((L, 1, D)), full((L, 1, D)),
            full((L, F, D)), full((L, 1, F)),
            full((L, D, F)), full((L, 1, D)),
            full((L, 1, D)), full((L, 1, D)),
            full((1, D)), full((1, D)),
        ],
        out_specs=pl.BlockSpec((PB, NTOK, D), lambda b: (b, 0, 0)),
        out_shape=jax.ShapeDtypeStruct((B, NTOK, D), jnp.float32),
        scratch_shapes=[pltpu.VMEM((PB * SP, D), jnp.float32)],
        compiler_params=pltpu.CompilerParams(
            vmem_limit_bytes=100 * 1024 * 1024),
    )(emb_t, emb_a, sv, Wqkv, bqkv, Wo, bo, ln1g, ln1b, W1, b1,
      W2, b2, ln2g, ln2b, lnfg, lnfb)


def kernel(token_ids, abund_bins, taxon_table, abund_table, sample_embed,
           Wqkv, bqkv, Wo, bo, ln1_g, ln1_b, W1, b1, W2, b2, ln2_g, ln2_b,
           lnf_g, lnf_b):
    emb_t, emb_a = _sc_gather(token_ids.reshape(N).astype(jnp.int32),
                              abund_bins.reshape(N).astype(jnp.int32),
                              taxon_table, abund_table)
    c = (DH ** -0.5) * 1.4426950408889634
    Wqkv = Wqkv.at[:, :D, :].multiply(c)
    bqkv = bqkv.at[:, :D].multiply(c)
    out = _tc_forward(
        emb_t.reshape(B, S, D), emb_a.reshape(B, S, D), sample_embed,
        Wqkv.astype(jnp.bfloat16), bqkv.reshape(L, 1, 3 * D),
        Wo.astype(jnp.bfloat16), bo.reshape(L, 1, D),
        ln1_g.reshape(L, 1, D), ln1_b.reshape(L, 1, D),
        W1.astype(jnp.bfloat16), b1.reshape(L, 1, F),
        W2.astype(jnp.bfloat16), b2.reshape(L, 1, D),
        ln2_g.reshape(L, 1, D), ln2_b.reshape(L, 1, D),
        lnf_g.reshape(1, D), lnf_b.reshape(1, D))
    return (out, out[:, 0, :])
